# baseline (speedup 1.0000x reference)
import functools

import jax
import jax.numpy as jnp
from jax import lax
from jax.experimental import pallas as pl
from jax.experimental.pallas import tpu as pltpu
from jax.experimental.pallas import tpu_sc as plsc

VOCAB = 100001
DIM = 64
BATCH = 16384
NF = 4
OUT_W = DIM + NF

NC = 2
NS = 16
NW = NC * NS
B_PER_W = BATCH // NW
L = 16
CHUNK = 128
N_CHUNKS = B_PER_W // CHUNK
F_WORDS = B_PER_W * NF
OUT_WORDS = B_PER_W * OUT_W

_MEAN = 0.5
_SCALE = float(1.0 / (1.0 / 12.0 + 1e-7) ** 0.5)

_mesh = plsc.VectorSubcoreMesh(core_axis_name="c", subcore_axis_name="s")


@functools.partial(
    pl.kernel,
    mesh=_mesh,
    out_type=jax.ShapeDtypeStruct((BATCH * OUT_W,), jnp.float32),
    scratch_types=[
        pltpu.VMEM((B_PER_W,), jnp.int32),
        pltpu.VMEM((B_PER_W, DIM), jnp.float32),
        pltpu.VMEM((OUT_WORDS + L,), jnp.float32),
        pltpu.VMEM((F_WORDS + L,), jnp.float32),
        pltpu.SemaphoreType.DMA,
    ],
    compiler_params=pltpu.CompilerParams(use_tc_tiling_on_sc=False),
)
def _sc_embed(uid_hbm, feats_hbm, table_hbm, out_hbm,
              idx_v, rows_v, out68_v, feats_v, sem):
    wid = lax.axis_index("s") * NC + lax.axis_index("c")
    base = wid * B_PER_W

    pltpu.sync_copy(uid_hbm.at[pl.ds(base, B_PER_W)], idx_v)
    handles = []
    for j in range(N_CHUNKS):
        handles.append(pltpu.async_copy(
            table_hbm.at[idx_v.at[pl.ds(j * CHUNK, CHUNK)]],
            rows_v.at[pl.ds(j * CHUNK, CHUNK)],
            sem,
        ))

    pltpu.sync_copy(feats_hbm.at[pl.ds(base * NF, F_WORDS)],
                    feats_v.at[pl.ds(0, F_WORDS)])
    for i in range(F_WORDS // L):
        feats_v[pl.ds(i * L, L)] = (
            feats_v[pl.ds(i * L, L)] - _MEAN) * _SCALE

    for h in handles:
        h.wait()

    @pl.loop(0, B_PER_W)
    def _assemble(r):
        off = r * OUT_W
        for c in range(DIM // L):
            out68_v[pl.ds(off + c * L, L)] = rows_v[r, pl.ds(c * L, L)]
        out68_v[pl.ds(off + DIM, L)] = feats_v[pl.ds(r * NF, L)]

    pltpu.sync_copy(out68_v.at[pl.ds(0, OUT_WORDS)],
                    out_hbm.at[pl.ds(base * OUT_W, OUT_WORDS)])


def kernel(user_id, review_day, review_month, review_year, review_weekday,
           table):
    uid = user_id.astype(jnp.int32)
    feats = jnp.concatenate(
        [review_day, review_month, review_year, review_weekday],
        axis=1).reshape(BATCH * NF)
    out = _sc_embed(uid, feats, table)
    return out.reshape(BATCH, OUT_W)


# speedup vs baseline: 1.0268x; 1.0268x over previous
import functools

import jax
import jax.numpy as jnp
from jax import lax
from jax.experimental import pallas as pl
from jax.experimental.pallas import tpu as pltpu
from jax.experimental.pallas import tpu_sc as plsc

VOCAB = 100001
DIM = 64
BATCH = 16384
NF = 4
OUT_W = DIM + NF

NC = 2
NS = 16
NW = NC * NS
B_PER_W = BATCH // NW
L = 16
CHUNK = 128
N_CHUNKS = B_PER_W // CHUNK
FPAD = 16
OUT_WORDS = B_PER_W * OUT_W

_MEAN = 0.5
_SCALE = float(1.0 / (1.0 / 12.0 + 1e-7) ** 0.5)

_mesh = plsc.VectorSubcoreMesh(core_axis_name="c", subcore_axis_name="s")


@functools.partial(
    pl.kernel,
    mesh=_mesh,
    out_type=jax.ShapeDtypeStruct((BATCH * OUT_W,), jnp.float32),
    scratch_types=[
        pltpu.VMEM((B_PER_W,), jnp.int32),
        pltpu.VMEM((B_PER_W, DIM), jnp.float32),
        pltpu.VMEM((OUT_WORDS + L,), jnp.float32),
        pltpu.VMEM((FPAD + B_PER_W + FPAD,), jnp.float32),
        pltpu.VMEM((FPAD + B_PER_W + FPAD,), jnp.float32),
        pltpu.VMEM((FPAD + B_PER_W + FPAD,), jnp.float32),
        pltpu.VMEM((FPAD + B_PER_W + FPAD,), jnp.float32),
        pltpu.SemaphoreType.DMA,
        pltpu.SemaphoreType.DMA,
    ],
    compiler_params=pltpu.CompilerParams(use_tc_tiling_on_sc=False),
)
def _sc_embed(uid_hbm, fd_hbm, fm_hbm, fy_hbm, fw_hbm, table_hbm, out_hbm,
              idx_v, rows_v, out68_v, f0, f1, f2, f3, gsem, osem):
    wid = lax.axis_index("s") * NC + lax.axis_index("c")
    base = wid * B_PER_W

    pltpu.sync_copy(uid_hbm.at[pl.ds(base, B_PER_W)], idx_v)
    gathers = []
    for j in range(N_CHUNKS):
        gathers.append(pltpu.async_copy(
            table_hbm.at[idx_v.at[pl.ds(j * CHUNK, CHUNK)]],
            rows_v.at[pl.ds(j * CHUNK, CHUNK)],
            gsem,
        ))

    for fbuf, src in ((f0, fd_hbm), (f1, fm_hbm), (f2, fy_hbm),
                      (f3, fw_hbm)):
        pltpu.sync_copy(src.at[pl.ds(base, B_PER_W)],
                        fbuf.at[pl.ds(FPAD, B_PER_W)])
    for fbuf in (f0, f1, f2, f3):
        for i in range((FPAD + B_PER_W) // L):
            fbuf[pl.ds(i * L, L)] = (fbuf[pl.ds(i * L, L)] - _MEAN) * _SCALE

    lanes = lax.iota(jnp.int32, L)
    m0, m1, m2 = lanes == 0, lanes == 1, lanes == 2

    writes = []
    for j in range(N_CHUNKS):
        gathers[j].wait()

        @pl.loop(j * CHUNK, (j + 1) * CHUNK)
        def _assemble(r):
            off = r * OUT_W
            for c in range(DIM // L):
                out68_v[pl.ds(off + c * L, L)] = rows_v[r, pl.ds(c * L, L)]
            t0 = f0[pl.ds(r + FPAD, L)]
            t1 = f1[pl.ds(r + FPAD - 1, L)]
            t2 = f2[pl.ds(r + FPAD - 2, L)]
            t3 = f3[pl.ds(r + FPAD - 3, L)]
            v = jnp.where(m0, t0, jnp.where(m1, t1, jnp.where(m2, t2, t3)))
            out68_v[pl.ds(off + DIM, L)] = v

        writes.append(pltpu.async_copy(
            out68_v.at[pl.ds(j * CHUNK * OUT_W, CHUNK * OUT_W)],
            out_hbm.at[pl.ds(base * OUT_W + j * CHUNK * OUT_W,
                             CHUNK * OUT_W)],
            osem,
        ))

    for w in writes:
        w.wait()


def kernel(user_id, review_day, review_month, review_year, review_weekday,
           table):
    uid = user_id.astype(jnp.int32)
    fd = review_day.reshape(BATCH)
    fm = review_month.reshape(BATCH)
    fy = review_year.reshape(BATCH)
    fw = review_weekday.reshape(BATCH)
    out = _sc_embed(uid, fd, fm, fy, fw, table)
    return out.reshape(BATCH, OUT_W)


# speedup vs baseline: 1.1803x; 1.1495x over previous
import functools

import jax
import jax.numpy as jnp
from jax import lax
from jax.experimental import pallas as pl
from jax.experimental.pallas import tpu as pltpu
from jax.experimental.pallas import tpu_sc as plsc

VOCAB = 100001
DIM = 64
BATCH = 16384
NF = 4
OUT_W = DIM + NF
PITCH = 128

NC = 2
NS = 16
NW = NC * NS
B_PER_W = BATCH // NW
L = 16
CHUNK = 128
N_CHUNKS = B_PER_W // CHUNK
FPAD = 16
OUT_WORDS = B_PER_W * PITCH

_MEAN = 0.5
_SCALE = float(1.0 / (1.0 / 12.0 + 1e-7) ** 0.5)

_mesh = plsc.VectorSubcoreMesh(core_axis_name="c", subcore_axis_name="s")


@functools.partial(
    pl.kernel,
    mesh=_mesh,
    out_type=jax.ShapeDtypeStruct((BATCH * PITCH,), jnp.float32),
    scratch_types=[
        pltpu.VMEM((B_PER_W,), jnp.int32),
        pltpu.VMEM((B_PER_W, DIM), jnp.float32),
        pltpu.VMEM((OUT_WORDS,), jnp.float32),
        pltpu.VMEM((FPAD + B_PER_W + FPAD,), jnp.float32),
        pltpu.VMEM((FPAD + B_PER_W + FPAD,), jnp.float32),
        pltpu.VMEM((FPAD + B_PER_W + FPAD,), jnp.float32),
        pltpu.VMEM((FPAD + B_PER_W + FPAD,), jnp.float32),
        pltpu.SemaphoreType.DMA,
        pltpu.SemaphoreType.DMA,
    ],
    compiler_params=pltpu.CompilerParams(use_tc_tiling_on_sc=False),
)
def _sc_embed(uid_hbm, fd_hbm, fm_hbm, fy_hbm, fw_hbm, table_hbm, out_hbm,
              idx_v, rows_v, out128_v, f0, f1, f2, f3, gsem, osem):
    wid = lax.axis_index("s") * NC + lax.axis_index("c")
    base = wid * B_PER_W

    pltpu.sync_copy(uid_hbm.at[pl.ds(base, B_PER_W)], idx_v)
    gathers = []
    for j in range(N_CHUNKS):
        gathers.append(pltpu.async_copy(
            table_hbm.at[idx_v.at[pl.ds(j * CHUNK, CHUNK)]],
            rows_v.at[pl.ds(j * CHUNK, CHUNK)],
            gsem,
        ))

    for fbuf, src in ((f0, fd_hbm), (f1, fm_hbm), (f2, fy_hbm),
                      (f3, fw_hbm)):
        pltpu.sync_copy(src.at[pl.ds(base, B_PER_W)],
                        fbuf.at[pl.ds(FPAD, B_PER_W)])
    for fbuf in (f0, f1, f2, f3):
        for i in range((FPAD + B_PER_W) // L):
            fbuf[pl.ds(i * L, L)] = (fbuf[pl.ds(i * L, L)] - _MEAN) * _SCALE

    lanes = lax.iota(jnp.int32, L)
    m0, m1, m2 = lanes == 0, lanes == 1, lanes == 2

    @pl.loop(0, B_PER_W)
    def _tails(r):
        t0 = f0[pl.ds(r + FPAD, L)]
        t1 = f1[pl.ds(r + FPAD - 1, L)]
        t2 = f2[pl.ds(r + FPAD - 2, L)]
        t3 = f3[pl.ds(r + FPAD - 3, L)]
        v = jnp.where(m0, t0, jnp.where(m1, t1, jnp.where(m2, t2, t3)))
        out128_v[pl.ds(r * PITCH + DIM, L)] = v

    writes = []
    for j in range(N_CHUNKS):
        gathers[j].wait()

        @pl.loop(j * CHUNK, (j + 1) * CHUNK)
        def _assemble(r):
            off = r * PITCH
            for c in range(DIM // L):
                out128_v[pl.ds(off + c * L, L)] = rows_v[r, pl.ds(c * L, L)]

        writes.append(pltpu.async_copy(
            out128_v.at[pl.ds(j * CHUNK * PITCH, CHUNK * PITCH)],
            out_hbm.at[pl.ds((base + j * CHUNK) * PITCH, CHUNK * PITCH)],
            osem,
        ))

    for w in writes:
        w.wait()


def kernel(user_id, review_day, review_month, review_year, review_weekday,
           table):
    uid = user_id.astype(jnp.int32)
    fd = review_day.reshape(BATCH)
    fm = review_month.reshape(BATCH)
    fy = review_year.reshape(BATCH)
    fw = review_weekday.reshape(BATCH)
    out = _sc_embed(uid, fd, fm, fy, fw, table)
    return out.reshape(BATCH, PITCH)[:, :OUT_W]
